# initial kernel scaffold (unmeasured)
import jax
import jax.numpy as jnp
from jax import lax
from jax.experimental import pallas as pl
from jax.experimental.pallas import tpu as pltpu

N_DEV = 8
SCALE = 64 ** -0.5


def kernel(Q, K, V):
    B, S, H, D = Q.shape

    Qt = jnp.transpose(Q, (0, 2, 1, 3)).astype(jnp.bfloat16)
    Kt = jnp.transpose(K, (0, 2, 1, 3)).astype(jnp.bfloat16)
    Vt = jnp.transpose(V, (0, 2, 1, 3)).astype(jnp.bfloat16)

    def body(q_ref, k_ref, v_ref, out_ref, buf, send_sems, recv_sems):
        my = lax.axis_index("i")
        left = lax.rem(my + N_DEV - 1, N_DEV)
        right = lax.rem(my + 1, N_DEV)

        barrier = pltpu.get_barrier_semaphore()
        for nbr in (left, right):
            pl.semaphore_signal(
                barrier, inc=1,
                device_id=(nbr,), device_id_type=pl.DeviceIdType.MESH,
            )
        pl.semaphore_wait(barrier, 2)

        buf[my, 0] = k_ref[...]
        buf[my, 1] = v_ref[...]

        for hop in range(N_DEV - 1):
            src = lax.rem(my - hop + N_DEV, N_DEV)
            rdma = pltpu.make_async_remote_copy(
                src_ref=buf.at[src],
                dst_ref=buf.at[src],
                send_sem=send_sems.at[hop],
                recv_sem=recv_sems.at[hop],
                device_id=(right,),
                device_id_type=pl.DeviceIdType.MESH,
            )
            rdma.start()
            rdma.wait()

        def compute_one(bh, carry):
            bi = bh // H
            hi = lax.rem(bh, H)
            q = q_ref[bi, hi]
            ss = []
            for c in range(N_DEV):
                kc = buf[c, 0, bi, hi]
                ss.append(
                    lax.dot_general(
                        q, kc,
                        dimension_numbers=(((1,), (1,)), ((), ())),
                        preferred_element_type=jnp.float32,
                    )
                )
            s = jnp.concatenate(ss, axis=1) * SCALE
            m = jnp.max(s, axis=1, keepdims=True)
            p = jnp.exp(s - m)
            l = jnp.sum(p, axis=1, keepdims=True)
            p = (p / l).astype(jnp.bfloat16)
            acc = jnp.zeros((S, D), jnp.float32)
            for c in range(N_DEV):
                vc = buf[c, 1, bi, hi]
                acc = acc + lax.dot_general(
                    p[:, c * S:(c + 1) * S], vc,
                    dimension_numbers=(((1,), (0,)), ((), ())),
                    preferred_element_type=jnp.float32,
                )
            out_ref[bi, hi] = acc
            return carry

        lax.fori_loop(0, B * H, compute_one, 0)

    out = pl.pallas_call(
        body,
        out_shape=jax.ShapeDtypeStruct((B, H, S, D), jnp.float32),
        in_specs=[pl.BlockSpec(memory_space=pltpu.VMEM)] * 3,
        out_specs=pl.BlockSpec(memory_space=pltpu.VMEM),
        scratch_shapes=[
            pltpu.VMEM((N_DEV, 2, B, H, S, D), jnp.bfloat16),
            pltpu.SemaphoreType.DMA((N_DEV - 1,)),
            pltpu.SemaphoreType.DMA((N_DEV - 1,)),
        ],
        compiler_params=pltpu.CompilerParams(collective_id=0),
    )(Qt, Kt, Vt)

    return jnp.transpose(out, (0, 2, 1, 3))


# baseline (device time: 439877 ns/iter reference)
import jax
import jax.numpy as jnp
from jax import lax
from jax.experimental import pallas as pl
from jax.experimental.pallas import tpu as pltpu

N_DEV = 8
SCALE = 64 ** -0.5


def kernel(Q, K, V):
    B, S, H, D = Q.shape

    Qt = jnp.transpose(Q, (0, 2, 1, 3)).astype(jnp.bfloat16)
    KVt = jnp.concatenate(
        [
            jnp.transpose(K, (0, 2, 1, 3)).astype(jnp.bfloat16),
            jnp.transpose(V, (0, 2, 1, 3)).astype(jnp.bfloat16),
        ],
        axis=-1,
    )

    def body(q_ref, kv_ref, out_ref, buf, send_sems, recv_sems):
        my = lax.axis_index("i")
        left = lax.rem(my + N_DEV - 1, N_DEV)
        right = lax.rem(my + 1, N_DEV)

        barrier = pltpu.get_barrier_semaphore()
        for nbr in (left, right):
            pl.semaphore_signal(
                barrier, inc=1,
                device_id=(nbr,), device_id_type=pl.DeviceIdType.MESH,
            )
        pl.semaphore_wait(barrier, 2)

        buf[my] = kv_ref[...]

        for hop in range(N_DEV - 1):
            src = lax.rem(my - hop + N_DEV, N_DEV)
            rdma = pltpu.make_async_remote_copy(
                src_ref=buf.at[src],
                dst_ref=buf.at[src],
                send_sem=send_sems.at[hop],
                recv_sem=recv_sems.at[hop],
                device_id=(right,),
                device_id_type=pl.DeviceIdType.MESH,
            )
            rdma.start()
            rdma.wait()

        def compute_one(bh, carry):
            bi = bh // H
            hi = lax.rem(bh, H)
            q = q_ref[bi, hi]
            ss = []
            for c in range(N_DEV):
                kc = buf[c, bi, hi, :, 0:D]
                ss.append(
                    lax.dot_general(
                        q, kc,
                        dimension_numbers=(((1,), (1,)), ((), ())),
                        preferred_element_type=jnp.float32,
                    )
                )
            s = jnp.concatenate(ss, axis=1) * SCALE
            m = jnp.max(s, axis=1, keepdims=True)
            p = jnp.exp(s - m)
            l = jnp.sum(p, axis=1, keepdims=True)
            p = (p / l).astype(jnp.bfloat16)
            acc = jnp.zeros((S, D), jnp.float32)
            for c in range(N_DEV):
                vc = buf[c, bi, hi, :, D:2 * D]
                acc = acc + lax.dot_general(
                    p[:, c * S:(c + 1) * S], vc,
                    dimension_numbers=(((1,), (0,)), ((), ())),
                    preferred_element_type=jnp.float32,
                )
            out_ref[bi, hi] = acc
            return carry

        lax.fori_loop(0, B * H, compute_one, 0)

    out = pl.pallas_call(
        body,
        out_shape=jax.ShapeDtypeStruct((B, H, S, D), jnp.float32),
        in_specs=[pl.BlockSpec(memory_space=pltpu.VMEM)] * 2,
        out_specs=pl.BlockSpec(memory_space=pltpu.VMEM),
        scratch_shapes=[
            pltpu.VMEM((N_DEV, B, H, S, 2 * D), jnp.bfloat16),
            pltpu.SemaphoreType.DMA((N_DEV - 1,)),
            pltpu.SemaphoreType.DMA((N_DEV - 1,)),
        ],
        compiler_params=pltpu.CompilerParams(
            collective_id=0,
            vmem_limit_bytes=48 * 1024 * 1024,
        ),
    )(Qt, KVt)

    return jnp.transpose(out, (0, 2, 1, 3))


# device time: 267143 ns/iter; 1.6466x vs baseline; 1.6466x over previous
import jax
import jax.numpy as jnp
from jax import lax
from jax.experimental import pallas as pl
from jax.experimental.pallas import tpu as pltpu

N_DEV = 8
SCALE = 64 ** -0.5


def kernel(Q, K, V):
    B, S, H, D = Q.shape

    Qt = jnp.transpose(Q, (0, 2, 3, 1)).astype(jnp.bfloat16)
    Kt = jnp.transpose(K, (0, 2, 3, 1)).astype(jnp.bfloat16)
    Vt = jnp.transpose(V, (0, 2, 3, 1)).astype(jnp.bfloat16)

    def body(q_ref, k_ref, v_ref, out_ref, kbuf, vbuf, ml_ref,
             ks_cw, kr_cw, vs_cw, vr_cw, ks_ccw, kr_ccw, vs_ccw, vr_ccw):
        my = lax.axis_index("i")
        left = lax.rem(my + N_DEV - 1, N_DEV)
        right = lax.rem(my + 1, N_DEV)

        def slot(off):
            return lax.rem(my + off + N_DEV, N_DEV)

        barrier = pltpu.get_barrier_semaphore()
        for nbr in (left, right):
            pl.semaphore_signal(
                barrier, inc=1,
                device_id=(nbr,), device_id_type=pl.DeviceIdType.MESH,
            )
        pl.semaphore_wait(barrier, 2)

        def mk(src, dst, ssem, rsem, dev):
            return pltpu.make_async_remote_copy(
                src_ref=src, dst_ref=dst, send_sem=ssem, recv_sem=rsem,
                device_id=(dev,), device_id_type=pl.DeviceIdType.MESH,
            )

        cw_k = [mk(k_ref, kbuf.at[slot(0)], ks_cw.at[0], kr_cw.at[0], right)]
        cw_v = [mk(v_ref, vbuf.at[slot(0)], vs_cw.at[0], vr_cw.at[0], right)]
        ccw_k = [mk(k_ref, kbuf.at[slot(0)], ks_ccw.at[0], kr_ccw.at[0], left)]
        ccw_v = [mk(v_ref, vbuf.at[slot(0)], vs_ccw.at[0], vr_ccw.at[0], left)]
        for r in range(2, 5):
            s_cw, s_ccw = slot(-(r - 1)), slot(r - 1)
            cw_k.append(mk(kbuf.at[s_cw], kbuf.at[s_cw],
                           ks_cw.at[r - 1], kr_cw.at[r - 1], right))
            ccw_v.append(mk(vbuf.at[s_ccw], vbuf.at[s_ccw],
                            vs_ccw.at[r - 1], vr_ccw.at[r - 1], left))
            if r < 4:
                cw_v.append(mk(vbuf.at[s_cw], vbuf.at[s_cw],
                               vs_cw.at[r - 1], vr_cw.at[r - 1], right))
                ccw_k.append(mk(kbuf.at[s_ccw], kbuf.at[s_ccw],
                                ks_ccw.at[r - 1], kr_ccw.at[r - 1], left))

        def process(get_k, get_v, first, last):
            def one(bh, carry):
                bi = bh // H
                hi = lax.rem(bh, H)
                qT = q_ref[bi, hi]
                kc = get_k(bi, hi)
                vc = get_v(bi, hi)
                sT = lax.dot_general(
                    kc, qT, (((0,), (0,)), ((), ())),
                    preferred_element_type=jnp.float32,
                ) * SCALE
                m_c = jnp.max(sT, axis=0, keepdims=True)
                if first:
                    m_new = m_c
                    pT = jnp.exp(sT - m_new)
                    l_new = jnp.sum(pT, axis=0, keepdims=True)
                    acc = lax.dot_general(
                        vc, pT.astype(jnp.bfloat16),
                        (((1,), (0,)), ((), ())),
                        preferred_element_type=jnp.float32,
                    )
                else:
                    m_old = ml_ref[bi, hi, 0:1, :]
                    l_old = ml_ref[bi, hi, 1:2, :]
                    m_new = jnp.maximum(m_old, m_c)
                    pT = jnp.exp(sT - m_new)
                    alpha = jnp.exp(m_old - m_new)
                    l_new = l_old * alpha + jnp.sum(pT, axis=0, keepdims=True)
                    acc = out_ref[bi, hi] * alpha + lax.dot_general(
                        vc, pT.astype(jnp.bfloat16),
                        (((1,), (0,)), ((), ())),
                        preferred_element_type=jnp.float32,
                    )
                if last:
                    out_ref[bi, hi] = acc / l_new
                else:
                    out_ref[bi, hi] = acc
                    ml_ref[bi, hi, 0:1, :] = m_new
                    ml_ref[bi, hi, 1:2, :] = l_new
                return carry
            lax.fori_loop(0, B * H, one, 0)

        def buf_getter(buf, s):
            return lambda bi, hi: buf[s, bi, hi]

        for d in (cw_k[0], cw_v[0], ccw_k[0], ccw_v[0]):
            d.start()
        process(lambda bi, hi: k_ref[bi, hi],
                lambda bi, hi: v_ref[bi, hi], first=True, last=False)

        for r in range(1, 4):
            cw_k[r - 1].wait_recv()
            if r < 4:
                cw_k[r].start()
            ccw_v[r - 1].wait_recv()
            if r < 4:
                ccw_v[r].start()
            cw_v[r - 1].wait_recv()
            if r < 3:
                cw_v[r].start()
            ccw_k[r - 1].wait_recv()
            if r < 3:
                ccw_k[r].start()
            process(buf_getter(kbuf, slot(-r)), buf_getter(vbuf, slot(-r)),
                    first=False, last=False)
            process(buf_getter(kbuf, slot(r)), buf_getter(vbuf, slot(r)),
                    first=False, last=False)

        cw_k[3].wait_recv()
        ccw_v[3].wait_recv()
        process(buf_getter(kbuf, slot(4)), buf_getter(vbuf, slot(4)),
                first=False, last=True)

        for d in cw_k + cw_v + ccw_k + ccw_v:
            d.wait_send()

    out = pl.pallas_call(
        body,
        out_shape=jax.ShapeDtypeStruct((B, H, D, S), jnp.float32),
        in_specs=[pl.BlockSpec(memory_space=pltpu.VMEM)] * 3,
        out_specs=pl.BlockSpec(memory_space=pltpu.VMEM),
        scratch_shapes=[
            pltpu.VMEM((N_DEV, B, H, D, S), jnp.bfloat16),
            pltpu.VMEM((N_DEV, B, H, D, S), jnp.bfloat16),
            pltpu.VMEM((B, H, 8, S), jnp.float32),
            pltpu.SemaphoreType.DMA((4,)),
            pltpu.SemaphoreType.DMA((4,)),
            pltpu.SemaphoreType.DMA((3,)),
            pltpu.SemaphoreType.DMA((3,)),
            pltpu.SemaphoreType.DMA((3,)),
            pltpu.SemaphoreType.DMA((3,)),
            pltpu.SemaphoreType.DMA((4,)),
            pltpu.SemaphoreType.DMA((4,)),
        ],
        compiler_params=pltpu.CompilerParams(
            collective_id=0,
            vmem_limit_bytes=56 * 1024 * 1024,
        ),
    )(Qt, Kt, Vt)

    return jnp.transpose(out, (0, 3, 1, 2))


# device time: 208529 ns/iter; 2.1094x vs baseline; 1.2811x over previous
import math

import jax
import jax.numpy as jnp
from jax import lax
from jax.experimental import pallas as pl
from jax.experimental.pallas import tpu as pltpu

N_DEV = 8
SCALE = 64 ** -0.5


def kernel(Q, K, V):
    B, S, H, D = Q.shape
    B2 = B // 2

    Qt = (jnp.transpose(Q, (0, 2, 1, 3)) * (SCALE * math.log2(math.e))
          ).astype(jnp.bfloat16)
    KVt = jnp.concatenate(
        [
            jnp.transpose(K, (0, 2, 1, 3)).astype(jnp.bfloat16),
            jnp.transpose(V, (0, 2, 1, 3)).astype(jnp.bfloat16),
        ],
        axis=-1,
    )

    def body(q_ref, kv_ref, out_ref, kvbuf, s_cw, r_cw, s_ccw, r_ccw):
        my = lax.axis_index("i")
        left = lax.rem(my + N_DEV - 1, N_DEV)
        right = lax.rem(my + 1, N_DEV)

        barrier = pltpu.get_barrier_semaphore()
        for nbr in (left, right):
            pl.semaphore_signal(
                barrier, inc=1,
                device_id=(nbr,), device_id_type=pl.DeviceIdType.MESH,
            )
        pl.semaphore_wait(barrier, 2)

        def mk(src, dst, ssem, rsem, dev):
            return pltpu.make_async_remote_copy(
                src_ref=src, dst_ref=dst, send_sem=ssem, recv_sem=rsem,
                device_id=(dev,), device_id_type=pl.DeviceIdType.MESH,
            )

        ROWS_L = pl.ds(0, S)
        ROWS_R = pl.ds(S, S)

        cw = [mk(kv_ref, kvbuf.at[0, :, :, ROWS_L, :],
                 s_cw.at[0], r_cw.at[0], right)]
        ccw = [mk(kv_ref, kvbuf.at[0, :, :, ROWS_R, :],
                  s_ccw.at[0], r_ccw.at[0], left)]
        for r in (1, 2):
            cw.append(mk(kvbuf.at[r - 1, :, :, ROWS_L, :],
                         kvbuf.at[r, :, :, ROWS_L, :],
                         s_cw.at[r], r_cw.at[r], right))
            ccw.append(mk(kvbuf.at[r - 1, :, :, ROWS_R, :],
                          kvbuf.at[r, :, :, ROWS_R, :],
                          s_ccw.at[r], r_ccw.at[r], left))
        cw.append(mk(kvbuf.at[2, pl.ds(0, B2), :, ROWS_L, :],
                     kvbuf.at[3, pl.ds(0, B2), :, ROWS_L, :],
                     s_cw.at[3], r_cw.at[3], right))
        ccw.append(mk(kvbuf.at[2, pl.ds(B2, B2), :, ROWS_R, :],
                      kvbuf.at[3, pl.ds(B2, B2), :, ROWS_R, :],
                      s_ccw.at[3], r_ccw.at[3], left))

        def process(get_kv, first=False, last=False, lo=0, hi=B * H):
            def one(bh, carry):
                bi = bh // H
                hi = lax.rem(bh, H)
                q = q_ref[bi, hi]
                kv = get_kv(bi, hi)
                s = lax.dot_general(
                    q, kv[:, 0:D], (((1,), (1,)), ((), ())),
                    preferred_element_type=jnp.float32,
                )
                p = jnp.exp2(s)
                l_add = jnp.sum(p, axis=1, keepdims=True)
                pv = lax.dot_general(
                    p.astype(jnp.bfloat16), kv[:, D:2 * D],
                    (((1,), (0,)), ((), ())),
                    preferred_element_type=jnp.float32,
                )
                if first:
                    acc, l_new = pv, l_add
                else:
                    prev = out_ref[bi, hi]
                    acc = prev[:, 0:D] + pv
                    l_new = prev[:, D:D + 1] + l_add
                if last:
                    acc = acc / l_new
                out_ref[bi, hi] = jnp.concatenate(
                    [acc, jnp.broadcast_to(l_new, (S, D))], axis=1)
                return carry
            lax.fori_loop(lo, hi, one, 0, unroll=8)

        for d in (cw[0], ccw[0]):
            d.start()
        process(lambda bi, hi: kv_ref[bi, hi], first=True)

        for r in (0, 1):
            cw[r].wait_recv()
            cw[r + 1].start()
            ccw[r].wait_recv()
            ccw[r + 1].start()
            process(lambda bi, hi, r=r: kvbuf[r, bi, hi])

        cw[2].wait_recv()
        cw[3].start()
        ccw[2].wait_recv()
        ccw[3].start()
        process(lambda bi, hi: kvbuf[2, bi, hi])

        cw[3].wait_recv()
        process(lambda bi, hi: kvbuf[3, bi, hi, ROWS_L], last=True,
                lo=0, hi=B2 * H)
        ccw[3].wait_recv()
        process(lambda bi, hi: kvbuf[3, bi, hi, ROWS_R], last=True,
                lo=B2 * H, hi=B * H)

        for d in cw + ccw:
            d.wait_send()

    out = pl.pallas_call(
        body,
        out_shape=jax.ShapeDtypeStruct((B, H, S, 2 * D), jnp.float32),
        in_specs=[pl.BlockSpec(memory_space=pltpu.VMEM)] * 2,
        out_specs=pl.BlockSpec(memory_space=pltpu.VMEM),
        scratch_shapes=[
            pltpu.VMEM((4, B, H, 2 * S, 2 * D), jnp.bfloat16),
            pltpu.SemaphoreType.DMA((4,)),
            pltpu.SemaphoreType.DMA((4,)),
            pltpu.SemaphoreType.DMA((4,)),
            pltpu.SemaphoreType.DMA((4,)),
        ],
        compiler_params=pltpu.CompilerParams(
            collective_id=0,
            vmem_limit_bytes=44 * 1024 * 1024,
        ),
    )(Qt, KVt)

    return jnp.transpose(out[..., 0:D], (0, 2, 1, 3))
